# baseline (speedup 1.0000x reference)
import functools

import numpy as np
import jax
import jax.numpy as jnp
from jax import lax
from jax.experimental import pallas as pl
from jax.experimental.pallas import tpu as pltpu
from jax.experimental.pallas import tpu_sc as plsc

N = 10000
EMB = 128
NC, NS = 2, 16
NW = NC * NS
DPT = 352
N_ACC = NW * DPT
CHUNK = 128
DUMMY = N_ACC - 1
RBLK = 2816
NLAYER = 5

_f32 = jnp.float32


def _h0_body(x0_hbm, x1_hbm, tab_hbm, out_hbm, idx_vm, buf, ident, ident2, acc_sh):
    c = lax.axis_index("c")
    s = lax.axis_index("s")
    tid = s * NC + c
    r0 = tid * DPT
    for k in range(8):
        ident[0, pl.ds(k * 16, 16)] = (
            lax.iota(jnp.int32, 16) + jnp.int32(k * 16))
    for base, sz in ((0, 128), (128, 128), (256, 96)):
        idx = idx_vm.at[0, pl.ds(0, sz)]
        pltpu.sync_copy(x0_hbm.at[pl.ds(r0 + base, sz)], idx)
        pltpu.sync_copy(tab_hbm.at[idx], buf.at[pl.ds(0, sz)])
        pltpu.sync_copy(buf.at[pl.ds(0, sz)],
                        acc_sh.at[pl.ds(r0 + base, sz)])
        pltpu.sync_copy(x1_hbm.at[pl.ds(r0 + base, sz)], idx)
        pltpu.sync_copy(tab_hbm.at[idx], buf.at[pl.ds(0, sz)])
        idref = ident if sz == 128 else ident2
        for k in range(sz // 16):
            idref[0, pl.ds(k * 16, 16)] = (
                lax.iota(jnp.int32, 16) + jnp.int32(r0 + base + k * 16))
        pltpu.sync_copy(buf.at[pl.ds(0, sz)], acc_sh.at[idref.at[0]],
                        add=True)
    pltpu.sync_copy(acc_sh.at[pl.ds(r0, DPT)], out_hbm.at[pl.ds(r0, DPT)])


def _agg_body(ch_all, h_hbm, src_hbm, dst_hbm, p_hbm, ptab_hbm, zeros_hbm,
              bnds_hbm, out_hbm, src_vm, dst_vm, p_vm, dstm, buf, ebuf,
              bnds_vm2, acc_sh):
    c = lax.axis_index("c")
    s = lax.axis_index("s")
    tid = s * NC + c
    r0 = tid * DPT
    dlo = r0
    dhi = r0 + DPT
    pltpu.sync_copy(bnds_hbm, bnds_vm2)
    pltpu.sync_copy(zeros_hbm.at[pl.ds(r0, DPT)], acc_sh.at[pl.ds(r0, DPT)])
    plsc.subcore_barrier()
    c0 = tid * 0
    c1 = jnp.int32(ch_all) + tid * 0

    def body(j, carry):
        @pl.when((j >= c0) & (j < c1))
        def _():
            _chunk(j)
        return carry

    def _chunk(j):
        e0 = j * CHUNK
        pltpu.sync_copy(src_hbm.at[pl.ds(e0, CHUNK)], src_vm.at[0])
        pltpu.sync_copy(dst_hbm.at[pl.ds(e0, CHUNK)], dst_vm.at[0])
        pltpu.sync_copy(p_hbm.at[pl.ds(e0, CHUNK)], p_vm.at[0])
        pltpu.sync_copy(h_hbm.at[src_vm.at[0]], buf)
        pltpu.sync_copy(ptab_hbm.at[p_vm.at[0]], ebuf)
        def addrow(r, cr):
            for k in range(8):
                sl = pl.ds(k * 16, 16)
                buf[r, sl] = buf[r, sl] + ebuf[r, sl]
            return cr
        lax.fori_loop(0, CHUNK, addrow, 0)
        for k in range(8):
            sl = pl.ds(k * 16, 16)
            d = dst_vm[0, sl]
            own = (d >= dlo) & (d < dhi)
            dstm[0, sl] = jnp.where(own, d, jnp.int32(DUMMY))
        pltpu.sync_copy(buf, acc_sh.at[dstm.at[0]], add=True)

    lax.fori_loop(0, ch_all, body, 0)
    plsc.subcore_barrier()
    pltpu.sync_copy(acc_sh.at[pl.ds(r0, DPT)], out_hbm.at[pl.ds(r0, DPT)])


_SC_MESH = plsc.VectorSubcoreMesh(core_axis_name="c", subcore_axis_name="s")

_h0 = pl.kernel(
    _h0_body,
    out_type=jax.ShapeDtypeStruct((N_ACC, EMB), _f32),
    mesh=_SC_MESH,
    scratch_types=[
        pltpu.VMEM((1, CHUNK), jnp.int32),
        pltpu.VMEM((CHUNK, EMB), _f32),
        pltpu.VMEM((1, CHUNK), jnp.int32),
        pltpu.VMEM((1, 96), jnp.int32),
        pltpu.VMEM_SHARED((N_ACC, EMB), _f32),
    ],
)


def _make_agg(ch_all):
    return pl.kernel(
        functools.partial(_agg_body, ch_all),
        out_type=jax.ShapeDtypeStruct((N_ACC, EMB), _f32),
        mesh=_SC_MESH,
        scratch_types=[
            pltpu.VMEM((1, CHUNK), jnp.int32),
            pltpu.VMEM((1, CHUNK), jnp.int32),
            pltpu.VMEM((1, CHUNK), jnp.int32),
            pltpu.VMEM((1, CHUNK), jnp.int32),
            pltpu.VMEM((CHUNK, EMB), _f32),
            pltpu.VMEM((CHUNK, EMB), _f32),
            pltpu.VMEM((NW, 16), jnp.int32),
            pltpu.VMEM_SHARED((N_ACC, EMB), _f32),
        ],
    )


def _mlp_body(s_ref, w1_ref, b1_ref, w2_ref, b2_ref, p_ref):
    agg = s_ref[...]
    hidden = jnp.maximum(
        jnp.dot(agg, w1_ref[...], preferred_element_type=_f32) + b1_ref[...],
        0.0)
    p_ref[...] = (jnp.dot(hidden, w2_ref[...], preferred_element_type=_f32)
                  + b2_ref[...])


def _bn_body(relu, p_ref, mean_ref, var_ref, gam_ref, bet_ref, o_ref):
    y = ((p_ref[...] - mean_ref[...]) / jnp.sqrt(var_ref[...] + 1e-5)
         * gam_ref[...] + bet_ref[...])
    if relu:
        y = jnp.maximum(y, 0.0)
    o_ref[...] = y


_GRID = N_ACC // RBLK

_mlp = pl.pallas_call(
    _mlp_body,
    grid=(_GRID,),
    in_specs=[pl.BlockSpec((RBLK, EMB), lambda i: (i, 0)),
              pl.BlockSpec((128, 256), lambda i: (0, 0)),
              pl.BlockSpec((1, 256), lambda i: (0, 0)),
              pl.BlockSpec((256, 128), lambda i: (0, 0)),
              pl.BlockSpec((1, 128), lambda i: (0, 0))],
    out_specs=pl.BlockSpec((RBLK, EMB), lambda i: (i, 0)),
    out_shape=jax.ShapeDtypeStruct((N_ACC, EMB), _f32),
)


def _make_bn(relu):
    return pl.pallas_call(
        functools.partial(_bn_body, relu),
        grid=(_GRID,),
        in_specs=[pl.BlockSpec((RBLK, EMB), lambda i: (i, 0)),
                  pl.BlockSpec((1, 128), lambda i: (0, 0)),
                  pl.BlockSpec((1, 128), lambda i: (0, 0)),
                  pl.BlockSpec((1, 128), lambda i: (0, 0)),
                  pl.BlockSpec((1, 128), lambda i: (0, 0))],
        out_specs=pl.BlockSpec((RBLK, EMB), lambda i: (i, 0)),
        out_shape=jax.ShapeDtypeStruct((N_ACC, EMB), _f32),
    )


_bn_relu = _make_bn(True)
_bn_last = _make_bn(False)


def kernel(x, edge_index, edge_attr, x_emb_table, edge_emb_tables,
           W1, b1, W2, b2, bn_gamma, bn_beta):
    E = edge_index.shape[1]
    loop = jnp.arange(N, dtype=jnp.int32)
    tot_src = jnp.concatenate([edge_index[0], loop])
    tot_dst = jnp.concatenate([edge_index[1], loop])
    tot_p = jnp.concatenate(
        [edge_attr[:, 0] * 3 + edge_attr[:, 1],
         jnp.full((N,), 9, jnp.int32)])
    order = jnp.argsort(tot_dst, stable=True)
    ssrc, sdst, sp = tot_src[order], tot_dst[order], tot_p[order]

    e_tot = E + N
    e_pad = ((e_tot + CHUNK - 1) // CHUNK) * CHUNK
    npad = e_pad - e_tot
    ssrc = jnp.concatenate([ssrc, jnp.zeros((npad,), jnp.int32)])
    sdst = jnp.concatenate([sdst, jnp.full((npad,), N, jnp.int32)])
    sp = jnp.concatenate([sp, jnp.full((npad,), 15, jnp.int32)])

    tile_lo = jnp.arange(NW + 1, dtype=jnp.int32) * DPT
    e_bnd = jnp.searchsorted(sdst, tile_lo, side="left").astype(jnp.int32)
    c0s = e_bnd[:-1] // CHUNK
    c1s = jnp.where(e_bnd[1:] > e_bnd[:-1],
                    (e_bnd[1:] + CHUNK - 1) // CHUNK, c0s)
    bnds = jnp.zeros((NW, 16), jnp.int32).at[:, 0].set(c0s).at[:, 1].set(c1s)

    x_pad = jnp.zeros((N_ACC, 2), jnp.int32).at[:N].set(x)
    x0 = x_pad[:, 0]
    x1 = x_pad[:, 1]
    table_pad = jnp.zeros((128, 128), _f32).at[:120].set(x_emb_table)
    zeros_emb = jnp.zeros((N_ACC, EMB), _f32)

    i_idx = jnp.asarray(np.array([p // 3 for p in range(9)] + [4],
                                 np.int32))
    j_idx = jnp.asarray(np.array([p % 3 for p in range(9)] + [0], np.int32))

    agg_fn = _make_agg(e_pad // CHUNK)

    h = _h0(x0, x1, table_pad)
    for l in range(NLAYER):
        tab = edge_emb_tables[l]
        ptab = jnp.zeros((16, 128), _f32).at[:10].set(
            tab[i_idx] + tab[j_idx])
        s_agg = agg_fn(h, ssrc, sdst, sp, ptab, zeros_emb, bnds)
        p = _mlp(s_agg, W1[l], b1[l].reshape(1, -1),
                 W2[l], b2[l].reshape(1, -1))
        pn = p[:N]
        mean = jnp.mean(pn, axis=0).reshape(1, -1)
        var = jnp.var(pn, axis=0).reshape(1, -1)
        bn = _bn_relu if l < NLAYER - 1 else _bn_last
        h = bn(p, mean, var, bn_gamma[l].reshape(1, -1),
               bn_beta[l].reshape(1, -1))
    return h[:N]
